# Initial kernel scaffold; baseline (speedup 1.0000x reference)
import jax
import jax.numpy as jnp
from jax.experimental import pallas as pl


def kernel(msa, pair, xyz, state, seq1hot, idx, top_k, params):
    raise NotImplementedError("write your pallas kernel here")


# speedup vs baseline: 6.2452x; 6.2452x over previous
import math

import jax
import jax.numpy as jnp
from jax.experimental import pallas as pl
from jax.experimental.pallas import tpu as pltpu

_N, _L = 64, 256
_DM, _DP, _DS = 64, 128, 16
_L0I, _L0O, _DE = 32, 16, 32
_HID = 64
_KMIN = 9.0
_BI = 32
_NB = _L // _BI
_RCH = 8


def _ln(x, g, b, eps=1e-5):
    mu = jnp.mean(x, axis=-1, keepdims=True)
    var = jnp.mean((x - mu) ** 2, axis=-1, keepdims=True)
    return (x - mu) / jnp.sqrt(var + eps) * g + b


def _mm(a, b):
    return jax.lax.dot_general(a, b, (((1,), (0,)), ((), ())),
                               preferred_element_type=jnp.float32)


def _mmT(a, b):
    return jax.lax.dot_general(a, b, (((0,), (0,)), ((), ())),
                               preferred_element_type=jnp.float32,
                               precision=jax.lax.Precision.HIGHEST)


def _body(pair_ref, msa_ref, xyz_ref, ca_ref, caT_ref, state_ref, seq_ref,
          idxc_ref, idxr_ref, kk_ref,
          g_msa, b_msa, Wq, bq, Wk, bk, g_state, b_state,
          Wx_m, Wx_s, Wx_t, bx, g_node, b_node,
          g_pair, b_pair, We1, be1, g_e1, b_e1,
          We2_p, We2_r, we2_n, be2, g_e2, b_e2,
          W1a, W1b, W1c, w1d, b1, W2, b2, Ws, Wg,
          state_out_ref, xyz_out_ref,
          A_s, Bt_s, Tacc, Racc):
    ib = pl.program_id(0)
    i0 = ib * _BI
    scale = 1.0 / math.sqrt(_DM)

    @pl.when(ib == 0)
    def _init():
        mflat = msa_ref[...].reshape(_N * _L, _DM)
        mln = _ln(mflat, g_msa[...], b_msa[...])
        mln3 = mln.reshape(_N, _L, _DM)
        q = _mm(mln3[0], Wq[...]) + bq[...]
        kmat = (_mm(mln, Wk[...]) + bk[...]).reshape(_N, _L, _DM)
        logits = jnp.sum(q[None, :, :] * scale * kmat, axis=2)
        mx = jnp.max(logits, axis=0, keepdims=True)
        ex = jnp.exp(logits - mx)
        attn = ex / jnp.sum(ex, axis=0, keepdims=True)
        msa1 = jnp.sum(attn[:, :, None] * mln3, axis=0)
        st_ln = _ln(state_ref[...], g_state[...], b_state[...])
        node = (_mm(msa1, Wx_m[...]) + _mm(seq_ref[...], Wx_s[...])
                + _mm(st_ln, Wx_t[...]) + bx[...])
        h0 = _ln(node, g_node[...], b_node[...])
        A_s[...] = _mm(h0, W1a[...]) + b1[...]
        Bt_s[...] = _mm(h0, W1b[...])
        state_out_ref[...] = jnp.zeros((_L, _L0O), jnp.float32)
        Tacc[...] = jnp.zeros((_L, 3), jnp.float32)
        Racc[...] = jnp.zeros((_L, 3), jnp.float32)

    ca = ca_ref[...]
    ca_i = ca_ref[pl.ds(i0, _BI), :]
    dx = ca_i[:, 0:1] - caT_ref[0:1, :]
    dy = ca_i[:, 1:2] - caT_ref[1:2, :]
    dz = ca_i[:, 2:3] - caT_ref[2:3, :]
    d2 = dx * dx + dy * dy + dz * dz
    D = jnp.sqrt(jnp.maximum(d2, 1e-12))

    row_ids = i0 + jax.lax.broadcasted_iota(jnp.int32, (_BI, _L), 0)
    col_ids = jax.lax.broadcasted_iota(jnp.int32, (_BI, _L), 1)
    eye = row_ids == col_ids
    De = D + jnp.where(eye, 999.9, 0.0)

    rank_rows = []
    for t in range(_BI // _RCH):
        r = De[t * _RCH:(t + 1) * _RCH]
        a = r[:, :, None]
        bb = r[:, None, :]
        jj = jax.lax.broadcasted_iota(jnp.int32, (_RCH, _L, _L), 1)
        jp = jax.lax.broadcasted_iota(jnp.int32, (_RCH, _L, _L), 2)
        cmp = (bb < a) | ((bb == a) & (jp < jj))
        rank_rows.append(jnp.sum(jnp.where(cmp, 1.0, 0.0), axis=2))
    ranks = jnp.concatenate(rank_rows, axis=0)

    seps = idxr_ref[...] - idxc_ref[pl.ds(i0, _BI), :]
    sepa = jnp.abs(seps) + jnp.where(eye, 999.9, 0.0)
    cond = (ranks < kk_ref[0, 0]) | (sepa < _KMIN)
    mask = jnp.where(cond, 1.0, 0.0)
    neigh = jnp.sign(seps) * jnp.where(jnp.abs(seps) > 1.0, 0.0, jnp.abs(seps))

    pr = pair_ref[...].reshape(_BI * _L, _DP)
    pln = _ln(pr, g_pair[...], b_pair[...])
    p1 = _ln(_mm(pln, We1[...]) + be1[...], g_e1[...], b_e1[...])
    mu = jax.lax.broadcasted_iota(
        jnp.int32, (1, 1, 36), 2).astype(jnp.float32) * (20.0 / 35.0)
    rbf = jnp.exp(-(((D[:, :, None] - mu) * (36.0 / 20.0)) ** 2))
    nterm = neigh[:, :, None] * we2_n[...][None]
    z = (_mm(p1, We2_p[...]) + _mm(rbf.reshape(_BI * _L, 36), We2_r[...])
         + nterm.reshape(_BI * _L, _DE) + be2[...])
    p2 = _ln(z, g_e2[...], b_e2[...])

    A_blk = A_s[pl.ds(i0, _BI), :]
    pre = (_mm(p2, W1c[...]).reshape(_BI, _L, _HID)
           + A_blk[:, None, :] + Bt_s[...][None, :, :]
           + D[:, :, None] * w1d[...][None])
    m1 = jnp.maximum(pre, 0.0).reshape(_BI * _L, _HID)
    m2 = jnp.maximum(_mm(m1, W2[...]) + b2[...], 0.0)

    s3 = (_mm(m2, Ws[...])).reshape(_BI, _L, _L0O) * mask[:, :, None]
    state_out_ref[...] += jnp.sum(s3, axis=0)

    gm = (_mm(m2, Wg[...])).reshape(_BI, _L, 8) * mask[:, :, None]
    ones_bi = jnp.ones((_BI, 1), jnp.float32)
    l1 = [xyz_ref[pl.ds(i0, _BI), k, :] - ca_i for k in range(3)]
    for c, acc_ref in ((0, Tacc), (1, Racc)):
        g0 = gm[:, :, 4 * c]
        S0 = _mmT(g0, ones_bi)
        acc = S0 * ca - _mmT(g0, ca_i)
        for k in range(3):
            acc = acc + _mmT(gm[:, :, 4 * c + 1 + k], l1[k])
        acc_ref[...] += acc

    @pl.when(ib == _NB - 1)
    def _fin():
        T = Tacc[...]
        R = Racc[...]
        R_angle = jnp.sqrt(jnp.sum(R * R, axis=1, keepdims=True))
        Rv = R / (R_angle + 1e-5)
        cosA = jnp.cos(R_angle)
        sinA = jnp.sin(R_angle)
        rx, ry, rz = Rv[:, 0:1], Rv[:, 1:2], Rv[:, 2:3]
        base = ca + T
        for k in range(3):
            vk = xyz_ref[:, k, :] - ca
            dk = jnp.sum(Rv * vk, axis=1, keepdims=True)
            vx, vy, vz = vk[:, 0:1], vk[:, 1:2], vk[:, 2:3]
            crossk = jnp.concatenate(
                [ry * vz - rz * vy, rz * vx - rx * vz, rx * vy - ry * vx],
                axis=1)
            u_par = Rv * dk
            v_new = (vk - u_par) * cosA + crossk * sinA + u_par
            xyz_out_ref[:, k, :] = v_new + base


def kernel(msa, pair, xyz, state, seq1hot, idx, top_k, params):
    p = params
    pair_r = pair[0]
    msa_r = msa[0]
    xyz_r = xyz[0]
    ca = xyz_r[:, 1, :]
    caT = ca.T
    state_r = state[0]
    seq_r = seq1hot[0]
    idxf = idx[0].astype(jnp.float32)
    idx_col = idxf.reshape(_L, 1)
    idx_row = idxf.reshape(1, _L)
    kk = jnp.minimum(jnp.asarray(top_k, jnp.float32),
                     jnp.float32(_L)).reshape(1, 1)

    def col(v, d):
        return v.reshape(1, d)

    W1 = p['W1']
    Wx = p['Wx']
    We2 = p['We2']
    weights = [
        col(p['g_msa'], _DM), col(p['b_msa'], _DM),
        p['Wq'], col(p['bq'], _DM), p['Wk'], col(p['bk'], _DM),
        col(p['g_state'], _DS), col(p['b_state'], _DS),
        Wx[:_DM], Wx[_DM:_DM + 21], Wx[_DM + 21:], col(p['bx'], _L0I),
        col(p['g_node'], _L0I), col(p['b_node'], _L0I),
        col(p['g_pair'], _DP), col(p['b_pair'], _DP),
        p['We1'], col(p['be1'], _DE), col(p['g_e1'], _DE), col(p['b_e1'], _DE),
        We2[:_DE], We2[_DE:_DE + 36], We2[_DE + 36:_DE + 37],
        col(p['be2'], _DE), col(p['g_e2'], _DE), col(p['b_e2'], _DE),
        W1[:_L0I], W1[_L0I:2 * _L0I], W1[2 * _L0I:2 * _L0I + _DE],
        W1[2 * _L0I + _DE:], col(p['b1'], _HID),
        p['W2'], col(p['b2'], _HID), p['Ws'], p['Wg'],
    ]

    data = [pair_r, msa_r, xyz_r, ca, caT, state_r, seq_r,
            idx_col, idx_row, kk]
    operands = data + weights

    in_specs = [pl.BlockSpec((_BI, _L, _DP), lambda i: (i, 0, 0))]
    for a in operands[1:]:
        in_specs.append(
            pl.BlockSpec(a.shape, lambda i, _nd=a.ndim: (0,) * _nd))

    out_specs = [
        pl.BlockSpec((_L, _L0O), lambda i: (0, 0)),
        pl.BlockSpec((_L, 3, 3), lambda i: (0, 0, 0)),
    ]
    out_shape = [
        jax.ShapeDtypeStruct((_L, _L0O), jnp.float32),
        jax.ShapeDtypeStruct((_L, 3, 3), jnp.float32),
    ]
    scratch_shapes = [
        pltpu.VMEM((_L, _HID), jnp.float32),
        pltpu.VMEM((_L, _HID), jnp.float32),
        pltpu.VMEM((_L, 3), jnp.float32),
        pltpu.VMEM((_L, 3), jnp.float32),
    ]

    state_out, xyz_out = pl.pallas_call(
        _body,
        grid=(_NB,),
        in_specs=in_specs,
        out_specs=out_specs,
        out_shape=out_shape,
        scratch_shapes=scratch_shapes,
        compiler_params=pltpu.CompilerParams(
            dimension_semantics=("arbitrary",)),
    )(*operands)

    return xyz_out.reshape(1, _L, 3, 3), state_out.reshape(1, _L, _L0O)


# speedup vs baseline: 6.4757x; 1.0369x over previous
import math

import jax
import jax.numpy as jnp
from jax.experimental import pallas as pl
from jax.experimental.pallas import tpu as pltpu

_N, _L = 64, 256
_DM, _DP, _DS = 64, 128, 16
_L0I, _L0O, _DE = 32, 16, 32
_HID = 64
_KMIN = 9.0
_BI = 32
_NB = _L // _BI
_RCH = 8


def _ln(x, g, b, eps=1e-5):
    mu = jnp.mean(x, axis=-1, keepdims=True)
    xc = x - mu
    var = jnp.mean(xc * xc, axis=-1, keepdims=True)
    inv = 1.0 / jnp.sqrt(var + eps)
    return xc * inv * g + b


def _mm(a, b):
    return jax.lax.dot_general(a, b, (((1,), (0,)), ((), ())),
                               preferred_element_type=jnp.float32)


def _mmT(a, b):
    return jax.lax.dot_general(a, b, (((0,), (0,)), ((), ())),
                               preferred_element_type=jnp.float32,
                               precision=jax.lax.Precision.HIGHEST)


def _body(pair_ref, msa_ref, xyz_ref, ca_ref, caT_ref, state_ref, seq_ref,
          idxc_ref, idxr_ref, kk_ref,
          g_msa, b_msa, Wq, bq, Wk, bk, g_state, b_state,
          Wx_m, Wx_s, Wx_t, bx, g_node, b_node,
          g_pair, b_pair, We1, be1, g_e1, b_e1,
          We2_p, We2_r, we2_n, be2, g_e2, b_e2,
          W1a, W1b, W1c, w1d, b1, W2, b2, WsG,
          state_out_ref, xyz_out_ref,
          A_s, Bt_s, Tacc, Racc):
    ib = pl.program_id(0)
    i0 = ib * _BI
    scale = 1.0 / math.sqrt(_DM)

    @pl.when(ib == 0)
    def _init():
        mflat = msa_ref[...].reshape(_N * _L, _DM)
        mln = _ln(mflat, g_msa[...], b_msa[...])
        mln3 = mln.reshape(_N, _L, _DM)
        q = _mm(mln3[0], Wq[...]) + bq[...]
        kmat = (_mm(mln, Wk[...]) + bk[...]).reshape(_N, _L, _DM)
        logits = jnp.sum(q[None, :, :] * scale * kmat, axis=2)
        mx = jnp.max(logits, axis=0, keepdims=True)
        ex = jnp.exp(logits - mx)
        attn = ex / jnp.sum(ex, axis=0, keepdims=True)
        msa1 = jnp.sum(attn[:, :, None] * mln3, axis=0)
        st_ln = _ln(state_ref[...], g_state[...], b_state[...])
        node = (_mm(msa1, Wx_m[...]) + _mm(seq_ref[...], Wx_s[...])
                + _mm(st_ln, Wx_t[...]) + bx[...])
        h0 = _ln(node, g_node[...], b_node[...])
        A_s[...] = _mm(h0, W1a[...]) + b1[...]
        Bt_s[...] = _mm(h0, W1b[...])
        state_out_ref[...] = jnp.zeros((_L, _L0O), jnp.float32)
        Tacc[...] = jnp.zeros((_L, 3), jnp.float32)
        Racc[...] = jnp.zeros((_L, 3), jnp.float32)

    ca = ca_ref[...]
    ca_i = ca_ref[pl.ds(i0, _BI), :]
    dx = ca_i[:, 0:1] - caT_ref[0:1, :]
    dy = ca_i[:, 1:2] - caT_ref[1:2, :]
    dz = ca_i[:, 2:3] - caT_ref[2:3, :]
    d2 = dx * dx + dy * dy + dz * dz
    D = jnp.sqrt(jnp.maximum(d2, 1e-12))

    row_ids = i0 + jax.lax.broadcasted_iota(jnp.int32, (_BI, _L), 0)
    col_ids = jax.lax.broadcasted_iota(jnp.int32, (_BI, _L), 1)
    eye = row_ids == col_ids
    De = D + jnp.where(eye, 999.9, 0.0)

    rank_rows = []
    for t in range(_BI // _RCH):
        r = De[t * _RCH:(t + 1) * _RCH]
        a = r[:, :, None]
        bb = r[:, None, :]
        jj = jax.lax.broadcasted_iota(jnp.int32, (_RCH, _L, _L), 1)
        jp = jax.lax.broadcasted_iota(jnp.int32, (_RCH, _L, _L), 2)
        cmp = (bb < a) | ((bb == a) & (jp < jj))
        rank_rows.append(jnp.sum(jnp.where(cmp, 1.0, 0.0), axis=2))
    ranks = jnp.concatenate(rank_rows, axis=0)

    seps = idxr_ref[...] - idxc_ref[pl.ds(i0, _BI), :]
    sepa = jnp.abs(seps) + jnp.where(eye, 999.9, 0.0)
    cond = (ranks < kk_ref[0, 0]) | (sepa < _KMIN)
    mask = jnp.where(cond, 1.0, 0.0)
    neigh = jnp.sign(seps) * jnp.where(jnp.abs(seps) > 1.0, 0.0, jnp.abs(seps))

    pr = pair_ref[...].reshape(_BI * _L, _DP)
    pln = _ln(pr, g_pair[...], b_pair[...])
    p1 = _ln(_mm(pln, We1[...]) + be1[...], g_e1[...], b_e1[...])
    mu = jax.lax.broadcasted_iota(
        jnp.int32, (1, 1, 36), 2).astype(jnp.float32) * (20.0 / 35.0)
    rbf = jnp.exp(-(((D[:, :, None] - mu) * (36.0 / 20.0)) ** 2))
    nterm = neigh[:, :, None] * we2_n[...][None]
    z = (_mm(p1, We2_p[...]) + _mm(rbf.reshape(_BI * _L, 36), We2_r[...])
         + nterm.reshape(_BI * _L, _DE) + be2[...])
    p2 = _ln(z, g_e2[...], b_e2[...])

    A_blk = A_s[pl.ds(i0, _BI), :]
    pre = (_mm(p2, W1c[...]).reshape(_BI, _L, _HID)
           + A_blk[:, None, :] + Bt_s[...][None, :, :]
           + D[:, :, None] * w1d[...][None])
    m1 = jnp.maximum(pre, 0.0).reshape(_BI * _L, _HID)
    m2 = jnp.maximum(_mm(m1, W2[...]) + b2[...], 0.0)

    sg = (_mm(m2, WsG[...])).reshape(_BI, _L, _L0O + 8) * mask[:, :, None]
    state_out_ref[...] += jnp.sum(sg[:, :, :_L0O], axis=0)
    gm = sg[:, :, _L0O:]
    ones_bi = jnp.ones((_BI, 1), jnp.float32)
    l1 = [xyz_ref[pl.ds(i0, _BI), k, :] - ca_i for k in range(3)]
    for c, acc_ref in ((0, Tacc), (1, Racc)):
        g0 = gm[:, :, 4 * c]
        S0 = _mmT(g0, ones_bi)
        acc = S0 * ca - _mmT(g0, ca_i)
        for k in range(3):
            acc = acc + _mmT(gm[:, :, 4 * c + 1 + k], l1[k])
        acc_ref[...] += acc

    @pl.when(ib == _NB - 1)
    def _fin():
        T = Tacc[...]
        R = Racc[...]
        R_angle = jnp.sqrt(jnp.sum(R * R, axis=1, keepdims=True))
        Rv = R / (R_angle + 1e-5)
        cosA = jnp.cos(R_angle)
        sinA = jnp.sin(R_angle)
        rx, ry, rz = Rv[:, 0:1], Rv[:, 1:2], Rv[:, 2:3]
        base = ca + T
        for k in range(3):
            vk = xyz_ref[:, k, :] - ca
            dk = jnp.sum(Rv * vk, axis=1, keepdims=True)
            vx, vy, vz = vk[:, 0:1], vk[:, 1:2], vk[:, 2:3]
            crossk = jnp.concatenate(
                [ry * vz - rz * vy, rz * vx - rx * vz, rx * vy - ry * vx],
                axis=1)
            u_par = Rv * dk
            v_new = (vk - u_par) * cosA + crossk * sinA + u_par
            xyz_out_ref[:, k, :] = v_new + base


def kernel(msa, pair, xyz, state, seq1hot, idx, top_k, params):
    p = params
    pair_r = pair[0]
    msa_r = msa[0]
    xyz_r = xyz[0]
    ca = xyz_r[:, 1, :]
    caT = ca.T
    state_r = state[0]
    seq_r = seq1hot[0]
    idxf = idx[0].astype(jnp.float32)
    idx_col = idxf.reshape(_L, 1)
    idx_row = idxf.reshape(1, _L)
    kk = jnp.minimum(jnp.asarray(top_k, jnp.float32),
                     jnp.float32(_L)).reshape(1, 1)

    def col(v, d):
        return v.reshape(1, d)

    W1 = p['W1']
    Wx = p['Wx']
    We2 = p['We2']
    weights = [
        col(p['g_msa'], _DM), col(p['b_msa'], _DM),
        p['Wq'], col(p['bq'], _DM), p['Wk'], col(p['bk'], _DM),
        col(p['g_state'], _DS), col(p['b_state'], _DS),
        Wx[:_DM], Wx[_DM:_DM + 21], Wx[_DM + 21:], col(p['bx'], _L0I),
        col(p['g_node'], _L0I), col(p['b_node'], _L0I),
        col(p['g_pair'], _DP), col(p['b_pair'], _DP),
        p['We1'], col(p['be1'], _DE), col(p['g_e1'], _DE), col(p['b_e1'], _DE),
        We2[:_DE], We2[_DE:_DE + 36], We2[_DE + 36:_DE + 37],
        col(p['be2'], _DE), col(p['g_e2'], _DE), col(p['b_e2'], _DE),
        W1[:_L0I], W1[_L0I:2 * _L0I], W1[2 * _L0I:2 * _L0I + _DE],
        W1[2 * _L0I + _DE:], col(p['b1'], _HID),
        p['W2'], col(p['b2'], _HID),
        jnp.concatenate([p['Ws'], p['Wg']], axis=1),
    ]

    data = [pair_r, msa_r, xyz_r, ca, caT, state_r, seq_r,
            idx_col, idx_row, kk]
    operands = data + weights

    in_specs = [pl.BlockSpec((_BI, _L, _DP), lambda i: (i, 0, 0))]
    for a in operands[1:]:
        in_specs.append(
            pl.BlockSpec(a.shape, lambda i, _nd=a.ndim: (0,) * _nd))

    out_specs = [
        pl.BlockSpec((_L, _L0O), lambda i: (0, 0)),
        pl.BlockSpec((_L, 3, 3), lambda i: (0, 0, 0)),
    ]
    out_shape = [
        jax.ShapeDtypeStruct((_L, _L0O), jnp.float32),
        jax.ShapeDtypeStruct((_L, 3, 3), jnp.float32),
    ]
    scratch_shapes = [
        pltpu.VMEM((_L, _HID), jnp.float32),
        pltpu.VMEM((_L, _HID), jnp.float32),
        pltpu.VMEM((_L, 3), jnp.float32),
        pltpu.VMEM((_L, 3), jnp.float32),
    ]

    state_out, xyz_out = pl.pallas_call(
        _body,
        grid=(_NB,),
        in_specs=in_specs,
        out_specs=out_specs,
        out_shape=out_shape,
        scratch_shapes=scratch_shapes,
        compiler_params=pltpu.CompilerParams(
            dimension_semantics=("arbitrary",)),
    )(*operands)

    return xyz_out.reshape(1, _L, 3, 3), state_out.reshape(1, _L, _L0O)


# speedup vs baseline: 9.4377x; 1.4574x over previous
import math

import jax
import jax.numpy as jnp
from jax.experimental import pallas as pl
from jax.experimental.pallas import tpu as pltpu

_N, _L = 64, 256
_DM, _DP, _DS = 64, 128, 16
_L0I, _L0O, _DE = 32, 16, 32
_HID = 64
_KMIN = 9.0
_BI = 16
_NB = _L // _BI
_RCH = 8


def _ln(x, g, b, eps=1e-5):
    mu = jnp.mean(x, axis=-1, keepdims=True)
    xc = x - mu
    var = jnp.mean(xc * xc, axis=-1, keepdims=True)
    inv = jax.lax.rsqrt(var + eps)
    return xc * inv * g + b


def _mm(a, b):
    return jax.lax.dot_general(a, b, (((1,), (0,)), ((), ())),
                               preferred_element_type=jnp.float32)


def _mmT(a, b):
    return jax.lax.dot_general(a, b, (((0,), (0,)), ((), ())),
                               preferred_element_type=jnp.float32,
                               precision=jax.lax.Precision.HIGHEST)


def _body(pair_ref, msa_ref, xyz_ref, ca_ref, caT_ref, state_ref, seq_ref,
          idxc_ref, idxr_ref, kk_ref,
          g_msa, b_msa, Wq, bq, Wk, bk, g_state, b_state,
          Wx_m, Wx_s, Wx_t, bx, g_node, b_node,
          g_pair, b_pair, We1, be1, g_e1, b_e1,
          We2_p, We2_r, we2_n, be2, g_e2, b_e2,
          W1a, W1b, W1c, w1d, b1, W2, b2, WsG,
          state_out_ref, xyz_out_ref,
          A_s, Bt_s, Tacc, Racc):
    ib = pl.program_id(0)
    i0 = ib * _BI
    scale = 1.0 / math.sqrt(_DM)

    @pl.when(ib == 0)
    def _init():
        mflat = msa_ref[...].reshape(_N * _L, _DM)
        mln = _ln(mflat, g_msa[...], b_msa[...])
        mln3 = mln.reshape(_N, _L, _DM)
        q = _mm(mln3[0], Wq[...]) + bq[...]
        kmat = (_mm(mln, Wk[...]) + bk[...]).reshape(_N, _L, _DM)
        logits = jnp.sum(q[None, :, :] * scale * kmat, axis=2)
        mx = jnp.max(logits, axis=0, keepdims=True)
        ex = jnp.exp(logits - mx)
        attn = ex / jnp.sum(ex, axis=0, keepdims=True)
        msa1 = jnp.sum(attn[:, :, None] * mln3, axis=0)
        st_ln = _ln(state_ref[...], g_state[...], b_state[...])
        node = (_mm(msa1, Wx_m[...]) + _mm(seq_ref[...], Wx_s[...])
                + _mm(st_ln, Wx_t[...]) + bx[...])
        h0 = _ln(node, g_node[...], b_node[...])
        A_s[...] = _mm(h0, W1a[...]) + b1[...]
        Bt_s[...] = _mm(h0, W1b[...])
        state_out_ref[...] = jnp.zeros((_L, _L0O), jnp.float32)
        Tacc[...] = jnp.zeros((_L, 3), jnp.float32)
        Racc[...] = jnp.zeros((_L, 3), jnp.float32)

    ca = ca_ref[...]
    ca_i = ca_ref[pl.ds(i0, _BI), :]
    dx = ca_i[:, 0:1] - caT_ref[0:1, :]
    dy = ca_i[:, 1:2] - caT_ref[1:2, :]
    dz = ca_i[:, 2:3] - caT_ref[2:3, :]
    d2 = dx * dx + dy * dy + dz * dz
    D = jnp.sqrt(jnp.maximum(d2, 1e-12))

    row_ids = i0 + jax.lax.broadcasted_iota(jnp.int32, (_BI, _L), 0)
    col_ids = jax.lax.broadcasted_iota(jnp.int32, (_BI, _L), 1)
    eye = row_ids == col_ids
    De = D + jnp.where(eye, 999.9, 0.0)

    rank_rows = []
    for t in range(_BI // _RCH):
        r = De[t * _RCH:(t + 1) * _RCH]
        a = r[:, :, None]
        bb = r[:, None, :]
        jj = jax.lax.broadcasted_iota(jnp.int32, (_RCH, _L, _L), 1)
        jp = jax.lax.broadcasted_iota(jnp.int32, (_RCH, _L, _L), 2)
        cmp = (bb < a) | ((bb == a) & (jp < jj))
        rank_rows.append(jnp.sum(jnp.where(cmp, 1.0, 0.0), axis=2,
                                 keepdims=True))
    ranks3 = jnp.concatenate(rank_rows, axis=0)

    dx3 = ca_i[:, 0:1, None] - ca[None, :, 0:1]
    dy3 = ca_i[:, 1:2, None] - ca[None, :, 1:2]
    dz3 = ca_i[:, 2:3, None] - ca[None, :, 2:3]
    D3 = jnp.sqrt(jnp.maximum(dx3 * dx3 + dy3 * dy3 + dz3 * dz3, 1e-12))

    row3 = i0 + jax.lax.broadcasted_iota(jnp.int32, (_BI, _L, 1), 0)
    col3 = jax.lax.broadcasted_iota(jnp.int32, (_BI, _L, 1), 1)
    eye3 = row3 == col3
    idx_i = idxc_ref[pl.ds(i0, _BI), :]
    seps3 = idxc_ref[...][None, :, :] - idx_i[:, :, None]
    sepa3 = jnp.abs(seps3) + jnp.where(eye3, 999.9, 0.0)
    cond3 = (ranks3 < kk_ref[0, 0]) | (sepa3 < _KMIN)
    mask3 = jnp.where(cond3, 1.0, 0.0)
    neigh3 = jnp.sign(seps3) * jnp.where(jnp.abs(seps3) > 1.0, 0.0,
                                         jnp.abs(seps3))

    pr = pair_ref[...].reshape(_BI * _L, _DP)
    pln = _ln(pr, g_pair[...], b_pair[...])
    p1 = _ln(_mm(pln, We1[...]) + be1[...], g_e1[...], b_e1[...])
    mu = jax.lax.broadcasted_iota(
        jnp.int32, (1, 1, 36), 2).astype(jnp.float32) * (20.0 / 35.0)
    rbf = jnp.exp(-(((D3 - mu) * (36.0 / 20.0)) ** 2))
    nterm = neigh3 * we2_n[...][None]
    z = (_mm(p1, We2_p[...]) + _mm(rbf.reshape(_BI * _L, 36), We2_r[...])
         + nterm.reshape(_BI * _L, _DE) + be2[...])
    p2 = _ln(z, g_e2[...], b_e2[...])

    A_blk = A_s[pl.ds(i0, _BI), :]
    pre = (_mm(p2, W1c[...]).reshape(_BI, _L, _HID)
           + A_blk[:, None, :] + Bt_s[...][None, :, :]
           + D3 * w1d[...][None])
    m1 = jnp.maximum(pre, 0.0).reshape(_BI * _L, _HID)
    m2 = jnp.maximum(_mm(m1, W2[...]) + b2[...], 0.0)

    sg = (_mm(m2, WsG[...])).reshape(_BI, _L, _L0O + 8) * mask3
    state_out_ref[...] += jnp.sum(sg[:, :, :_L0O], axis=0)
    gm = sg[:, :, _L0O:]
    ones_bi = jnp.ones((_BI, 1), jnp.float32)
    l1 = [xyz_ref[pl.ds(i0, _BI), k, :] - ca_i for k in range(3)]
    for c, acc_ref in ((0, Tacc), (1, Racc)):
        g0 = gm[:, :, 4 * c]
        S0 = _mmT(g0, ones_bi)
        acc = S0 * ca - _mmT(g0, ca_i)
        for k in range(3):
            acc = acc + _mmT(gm[:, :, 4 * c + 1 + k], l1[k])
        acc_ref[...] += acc

    @pl.when(ib == _NB - 1)
    def _fin():
        T = Tacc[...]
        R = Racc[...]
        R_angle = jnp.sqrt(jnp.sum(R * R, axis=1, keepdims=True))
        Rv = R / (R_angle + 1e-5)
        cosA = jnp.cos(R_angle)
        sinA = jnp.sin(R_angle)
        rx, ry, rz = Rv[:, 0:1], Rv[:, 1:2], Rv[:, 2:3]
        base = ca + T
        for k in range(3):
            vk = xyz_ref[:, k, :] - ca
            dk = jnp.sum(Rv * vk, axis=1, keepdims=True)
            vx, vy, vz = vk[:, 0:1], vk[:, 1:2], vk[:, 2:3]
            crossk = jnp.concatenate(
                [ry * vz - rz * vy, rz * vx - rx * vz, rx * vy - ry * vx],
                axis=1)
            u_par = Rv * dk
            v_new = (vk - u_par) * cosA + crossk * sinA + u_par
            xyz_out_ref[:, k, :] = v_new + base


def kernel(msa, pair, xyz, state, seq1hot, idx, top_k, params):
    p = params
    pair_r = pair[0]
    msa_r = msa[0]
    xyz_r = xyz[0]
    ca = xyz_r[:, 1, :]
    caT = ca.T
    state_r = state[0]
    seq_r = seq1hot[0]
    idxf = idx[0].astype(jnp.float32)
    idx_col = idxf.reshape(_L, 1)
    idx_row = idxf.reshape(1, _L)
    kk = jnp.minimum(jnp.asarray(top_k, jnp.float32),
                     jnp.float32(_L)).reshape(1, 1)

    def col(v, d):
        return v.reshape(1, d)

    W1 = p['W1']
    Wx = p['Wx']
    We2 = p['We2']
    weights = [
        col(p['g_msa'], _DM), col(p['b_msa'], _DM),
        p['Wq'], col(p['bq'], _DM), p['Wk'], col(p['bk'], _DM),
        col(p['g_state'], _DS), col(p['b_state'], _DS),
        Wx[:_DM], Wx[_DM:_DM + 21], Wx[_DM + 21:], col(p['bx'], _L0I),
        col(p['g_node'], _L0I), col(p['b_node'], _L0I),
        col(p['g_pair'], _DP), col(p['b_pair'], _DP),
        p['We1'], col(p['be1'], _DE), col(p['g_e1'], _DE), col(p['b_e1'], _DE),
        We2[:_DE], We2[_DE:_DE + 36], We2[_DE + 36:_DE + 37],
        col(p['be2'], _DE), col(p['g_e2'], _DE), col(p['b_e2'], _DE),
        W1[:_L0I], W1[_L0I:2 * _L0I], W1[2 * _L0I:2 * _L0I + _DE],
        W1[2 * _L0I + _DE:], col(p['b1'], _HID),
        p['W2'], col(p['b2'], _HID),
        jnp.concatenate([p['Ws'], p['Wg']], axis=1),
    ]

    data = [pair_r, msa_r, xyz_r, ca, caT, state_r, seq_r,
            idx_col, idx_row, kk]
    operands = data + weights

    in_specs = [pl.BlockSpec((_BI, _L, _DP), lambda i: (i, 0, 0))]
    for a in operands[1:]:
        in_specs.append(
            pl.BlockSpec(a.shape, lambda i, _nd=a.ndim: (0,) * _nd))

    out_specs = [
        pl.BlockSpec((_L, _L0O), lambda i: (0, 0)),
        pl.BlockSpec((_L, 3, 3), lambda i: (0, 0, 0)),
    ]
    out_shape = [
        jax.ShapeDtypeStruct((_L, _L0O), jnp.float32),
        jax.ShapeDtypeStruct((_L, 3, 3), jnp.float32),
    ]
    scratch_shapes = [
        pltpu.VMEM((_L, _HID), jnp.float32),
        pltpu.VMEM((_L, _HID), jnp.float32),
        pltpu.VMEM((_L, 3), jnp.float32),
        pltpu.VMEM((_L, 3), jnp.float32),
    ]

    state_out, xyz_out = pl.pallas_call(
        _body,
        grid=(_NB,),
        in_specs=in_specs,
        out_specs=out_specs,
        out_shape=out_shape,
        scratch_shapes=scratch_shapes,
        compiler_params=pltpu.CompilerParams(
            dimension_semantics=("arbitrary",)),
    )(*operands)

    return xyz_out.reshape(1, _L, 3, 3), state_out.reshape(1, _L, _L0O)


# speedup vs baseline: 9.4442x; 1.0007x over previous
import math

import jax
import jax.numpy as jnp
from jax.experimental import pallas as pl
from jax.experimental.pallas import tpu as pltpu

_N, _L = 64, 256
_DM, _DP, _DS = 64, 128, 16
_L0I, _L0O, _DE = 32, 16, 32
_HID = 64
_KMIN = 9.0
_BI = 16
_NB = _L // _BI
_RCH = 8


def _ln(x, g, b, eps=1e-5):
    mu = jnp.mean(x, axis=-1, keepdims=True)
    xc = x - mu
    var = jnp.mean(xc * xc, axis=-1, keepdims=True)
    inv = jax.lax.rsqrt(var + eps)
    return xc * inv * g + b


def _mm(a, b):
    return jax.lax.dot_general(a, b, (((1,), (0,)), ((), ())),
                               preferred_element_type=jnp.float32)


def _mm3(a, b):
    return jax.lax.dot_general(a, b, (((2,), (0,)), ((), ())),
                               preferred_element_type=jnp.float32)


def _mmT(a, b):
    return jax.lax.dot_general(a, b, (((0,), (0,)), ((), ())),
                               preferred_element_type=jnp.float32,
                               precision=jax.lax.Precision.HIGHEST)


def _body(pair_ref, msa_ref, xyz_ref, ca_ref, caT_ref, state_ref, seq_ref,
          idxc_ref, idxr_ref, kk_ref,
          g_msa, b_msa, Wq, bq, Wk, bk, g_state, b_state,
          Wx_m, Wx_s, Wx_t, bx, g_node, b_node,
          g_pair, b_pair, We1, be1, g_e1, b_e1,
          We2_p, We2_r, we2_n, be2, g_e2, b_e2,
          W1a, W1b, W1c, w1d, b1, W2, b2, WsG,
          state_out_ref, xyz_out_ref,
          A_s, Bt_s, Tacc, Racc):
    ib = pl.program_id(0)
    i0 = ib * _BI
    scale = 1.0 / math.sqrt(_DM)

    @pl.when(ib == 0)
    def _init():
        mflat = msa_ref[...].reshape(_N * _L, _DM)
        mln = _ln(mflat, g_msa[...], b_msa[...])
        mln3 = mln.reshape(_N, _L, _DM)
        q = _mm(mln3[0], Wq[...]) + bq[...]
        kmat = (_mm(mln, Wk[...]) + bk[...]).reshape(_N, _L, _DM)
        logits = jnp.sum(q[None, :, :] * scale * kmat, axis=2)
        mx = jnp.max(logits, axis=0, keepdims=True)
        ex = jnp.exp(logits - mx)
        attn = ex / jnp.sum(ex, axis=0, keepdims=True)
        msa1 = jnp.sum(attn[:, :, None] * mln3, axis=0)
        st_ln = _ln(state_ref[...], g_state[...], b_state[...])
        node = (_mm(msa1, Wx_m[...]) + _mm(seq_ref[...], Wx_s[...])
                + _mm(st_ln, Wx_t[...]) + bx[...])
        h0 = _ln(node, g_node[...], b_node[...])
        A_s[...] = _mm(h0, W1a[...]) + b1[...]
        Bt_s[...] = _mm(h0, W1b[...])
        state_out_ref[...] = jnp.zeros((_L, _L0O), jnp.float32)
        Tacc[...] = jnp.zeros((_L, 3), jnp.float32)
        Racc[...] = jnp.zeros((_L, 3), jnp.float32)

    ca = ca_ref[...]
    ca_i = ca_ref[pl.ds(i0, _BI), :]
    dx = ca_i[:, 0:1] - caT_ref[0:1, :]
    dy = ca_i[:, 1:2] - caT_ref[1:2, :]
    dz = ca_i[:, 2:3] - caT_ref[2:3, :]
    d2 = dx * dx + dy * dy + dz * dz
    D = jnp.sqrt(jnp.maximum(d2, 1e-12))

    row_ids = i0 + jax.lax.broadcasted_iota(jnp.int32, (_BI, _L), 0)
    col_ids = jax.lax.broadcasted_iota(jnp.int32, (_BI, _L), 1)
    eye = row_ids == col_ids
    De = D + jnp.where(eye, 999.9, 0.0)

    rank_rows = []
    for t in range(_BI // _RCH):
        r = De[t * _RCH:(t + 1) * _RCH]
        a = r[:, :, None]
        bb = r[:, None, :]
        jj = jax.lax.broadcasted_iota(jnp.int32, (_RCH, _L, _L), 1)
        jp = jax.lax.broadcasted_iota(jnp.int32, (_RCH, _L, _L), 2)
        cmp = (bb < a) | ((bb == a) & (jp < jj))
        rank_rows.append(jnp.sum(jnp.where(cmp, 1.0, 0.0), axis=2,
                                 keepdims=True))
    ranks3 = jnp.concatenate(rank_rows, axis=0)

    dx3 = ca_i[:, 0:1, None] - ca[None, :, 0:1]
    dy3 = ca_i[:, 1:2, None] - ca[None, :, 1:2]
    dz3 = ca_i[:, 2:3, None] - ca[None, :, 2:3]
    D3 = jnp.sqrt(jnp.maximum(dx3 * dx3 + dy3 * dy3 + dz3 * dz3, 1e-12))

    row3 = i0 + jax.lax.broadcasted_iota(jnp.int32, (_BI, _L, 1), 0)
    col3 = jax.lax.broadcasted_iota(jnp.int32, (_BI, _L, 1), 1)
    eye3 = row3 == col3
    idx_i = idxc_ref[pl.ds(i0, _BI), :]
    seps3 = idxc_ref[...][None, :, :] - idx_i[:, :, None]
    sepa3 = jnp.abs(seps3) + jnp.where(eye3, 999.9, 0.0)
    cond3 = (ranks3 < kk_ref[0, 0]) | (sepa3 < _KMIN)
    mask3 = jnp.where(cond3, 1.0, 0.0)
    neigh3 = jnp.sign(seps3) * jnp.where(jnp.abs(seps3) > 1.0, 0.0,
                                         jnp.abs(seps3))

    pln = _ln(pair_ref[...], g_pair[...], b_pair[...])
    p1 = _ln(_mm3(pln, We1[...]) + be1[...], g_e1[...], b_e1[...])
    mu = jax.lax.broadcasted_iota(
        jnp.int32, (1, 1, 36), 2).astype(jnp.float32) * (20.0 / 35.0)
    rbf = jnp.exp(-(((D3 - mu) * (36.0 / 20.0)) ** 2))
    nterm = neigh3 * we2_n[...][None]
    z = _mm3(p1, We2_p[...]) + _mm3(rbf, We2_r[...]) + nterm + be2[...]
    p2 = _ln(z, g_e2[...], b_e2[...])

    A_blk = A_s[pl.ds(i0, _BI), :]
    pre = (_mm3(p2, W1c[...])
           + A_blk[:, None, :] + Bt_s[...][None, :, :]
           + D3 * w1d[...][None])
    m1 = jnp.maximum(pre, 0.0)
    m2 = jnp.maximum(_mm3(m1, W2[...]) + b2[...], 0.0)

    sg = _mm3(m2, WsG[...]) * mask3
    state_out_ref[...] += jnp.sum(sg[:, :, :_L0O], axis=0)
    gm = sg[:, :, _L0O:]
    ones_bi = jnp.ones((_BI, 1), jnp.float32)
    l1 = [xyz_ref[pl.ds(i0, _BI), k, :] - ca_i for k in range(3)]
    for c, acc_ref in ((0, Tacc), (1, Racc)):
        g0 = gm[:, :, 4 * c]
        S0 = _mmT(g0, ones_bi)
        acc = S0 * ca - _mmT(g0, ca_i)
        for k in range(3):
            acc = acc + _mmT(gm[:, :, 4 * c + 1 + k], l1[k])
        acc_ref[...] += acc

    @pl.when(ib == _NB - 1)
    def _fin():
        T = Tacc[...]
        R = Racc[...]
        R_angle = jnp.sqrt(jnp.sum(R * R, axis=1, keepdims=True))
        Rv = R / (R_angle + 1e-5)
        cosA = jnp.cos(R_angle)
        sinA = jnp.sin(R_angle)
        rx, ry, rz = Rv[:, 0:1], Rv[:, 1:2], Rv[:, 2:3]
        base = ca + T
        for k in range(3):
            vk = xyz_ref[:, k, :] - ca
            dk = jnp.sum(Rv * vk, axis=1, keepdims=True)
            vx, vy, vz = vk[:, 0:1], vk[:, 1:2], vk[:, 2:3]
            crossk = jnp.concatenate(
                [ry * vz - rz * vy, rz * vx - rx * vz, rx * vy - ry * vx],
                axis=1)
            u_par = Rv * dk
            v_new = (vk - u_par) * cosA + crossk * sinA + u_par
            xyz_out_ref[:, k, :] = v_new + base


def kernel(msa, pair, xyz, state, seq1hot, idx, top_k, params):
    p = params
    pair_r = pair[0]
    msa_r = msa[0]
    xyz_r = xyz[0]
    ca = xyz_r[:, 1, :]
    caT = ca.T
    state_r = state[0]
    seq_r = seq1hot[0]
    idxf = idx[0].astype(jnp.float32)
    idx_col = idxf.reshape(_L, 1)
    idx_row = idxf.reshape(1, _L)
    kk = jnp.minimum(jnp.asarray(top_k, jnp.float32),
                     jnp.float32(_L)).reshape(1, 1)

    def col(v, d):
        return v.reshape(1, d)

    W1 = p['W1']
    Wx = p['Wx']
    We2 = p['We2']
    weights = [
        col(p['g_msa'], _DM), col(p['b_msa'], _DM),
        p['Wq'], col(p['bq'], _DM), p['Wk'], col(p['bk'], _DM),
        col(p['g_state'], _DS), col(p['b_state'], _DS),
        Wx[:_DM], Wx[_DM:_DM + 21], Wx[_DM + 21:], col(p['bx'], _L0I),
        col(p['g_node'], _L0I), col(p['b_node'], _L0I),
        col(p['g_pair'], _DP), col(p['b_pair'], _DP),
        p['We1'], col(p['be1'], _DE), col(p['g_e1'], _DE), col(p['b_e1'], _DE),
        We2[:_DE], We2[_DE:_DE + 36], We2[_DE + 36:_DE + 37],
        col(p['be2'], _DE), col(p['g_e2'], _DE), col(p['b_e2'], _DE),
        W1[:_L0I], W1[_L0I:2 * _L0I], W1[2 * _L0I:2 * _L0I + _DE],
        W1[2 * _L0I + _DE:], col(p['b1'], _HID),
        p['W2'], col(p['b2'], _HID),
        jnp.concatenate([p['Ws'], p['Wg']], axis=1),
    ]

    data = [pair_r, msa_r, xyz_r, ca, caT, state_r, seq_r,
            idx_col, idx_row, kk]
    operands = data + weights

    in_specs = [pl.BlockSpec((_BI, _L, _DP), lambda i: (i, 0, 0))]
    for a in operands[1:]:
        in_specs.append(
            pl.BlockSpec(a.shape, lambda i, _nd=a.ndim: (0,) * _nd))

    out_specs = [
        pl.BlockSpec((_L, _L0O), lambda i: (0, 0)),
        pl.BlockSpec((_L, 3, 3), lambda i: (0, 0, 0)),
    ]
    out_shape = [
        jax.ShapeDtypeStruct((_L, _L0O), jnp.float32),
        jax.ShapeDtypeStruct((_L, 3, 3), jnp.float32),
    ]
    scratch_shapes = [
        pltpu.VMEM((_L, _HID), jnp.float32),
        pltpu.VMEM((_L, _HID), jnp.float32),
        pltpu.VMEM((_L, 3), jnp.float32),
        pltpu.VMEM((_L, 3), jnp.float32),
    ]

    state_out, xyz_out = pl.pallas_call(
        _body,
        grid=(_NB,),
        in_specs=in_specs,
        out_specs=out_specs,
        out_shape=out_shape,
        scratch_shapes=scratch_shapes,
        compiler_params=pltpu.CompilerParams(
            dimension_semantics=("arbitrary",)),
    )(*operands)

    return xyz_out.reshape(1, _L, 3, 3), state_out.reshape(1, _L, _L0O)
